# baseline (speedup 1.0000x reference)
import functools

import jax
import jax.numpy as jnp
from jax.experimental import pallas as pl
from jax.experimental.pallas import tpu as pltpu
from jax.sharding import PartitionSpec as P

LEAK = 0.6
BIAS = 1.6

_N_TILE = 512


def _esn_body(off_ref, u_ref, s_ref, wr_ref, o_ref, s_bf16_ref):
    j = pl.program_id(0)

    @pl.when(j == 0)
    def _():
        s_bf16_ref[...] = s_ref[...].astype(jnp.bfloat16)

    pre = jax.lax.dot_general(
        s_bf16_ref[...],
        wr_ref[...].astype(jnp.bfloat16),
        dimension_numbers=(((1,), (1,)), ((), ())),
        preferred_element_type=jnp.float32,
    )
    pre = pre + u_ref[...] + BIAS
    col0 = pl.multiple_of(off_ref[0] + j * _N_TILE, _N_TILE)
    s_tile = s_ref[:, pl.ds(col0, _N_TILE)]
    o_ref[...] = LEAK * jnp.tanh(pre) + (1.0 - LEAK) * s_tile


def _esn_local(u_loc, s_full, wr_loc, col_off):
    batch, res_dim = s_full.shape
    n_loc = wr_loc.shape[0]
    n_tiles = n_loc // _N_TILE
    return pl.pallas_call(
        _esn_body,
        grid=(n_tiles,),
        in_specs=[
            pl.BlockSpec(memory_space=pltpu.SMEM),
            pl.BlockSpec((batch, _N_TILE), lambda j: (0, j)),
            pl.BlockSpec((batch, res_dim), lambda j: (0, 0)),
            pl.BlockSpec((_N_TILE, res_dim), lambda j: (j, 0)),
        ],
        out_specs=pl.BlockSpec((batch, _N_TILE), lambda j: (0, j)),
        out_shape=jax.ShapeDtypeStruct((batch, n_loc), jnp.float32),
        scratch_shapes=[pltpu.VMEM((batch, res_dim), jnp.bfloat16)],
    )(col_off, u_loc, s_full, wr_loc)


@jax.jit
def kernel(proj_vars, res_state, wr):
    res_dim = wr.shape[0]
    n_dev = len(jax.devices())
    while n_dev > 1 and (res_dim // _N_TILE) % n_dev:
        n_dev -= 1
    if n_dev == 1:
        zero = jnp.zeros((1,), jnp.int32)
        return _esn_local(proj_vars, res_state, wr, zero)

    mesh = jax.make_mesh((n_dev,), ("n",))
    proj_vars = jax.reshard(proj_vars, jax.NamedSharding(mesh, P(None, "n")))
    res_state = jax.reshard(res_state, jax.NamedSharding(mesh, P(None, None)))
    wr = jax.reshard(wr, jax.NamedSharding(mesh, P("n", None)))

    def _shard_fn(u_loc, s_full, wr_loc):
        d = jax.lax.axis_index("n")
        off = (d * (res_dim // n_dev)).astype(jnp.int32).reshape((1,))
        return _esn_local(u_loc, s_full, wr_loc, off)

    fn = jax.shard_map(
        _shard_fn,
        mesh=mesh,
        in_specs=(P(None, "n"), P(None, None), P("n", None)),
        out_specs=P(None, "n"),
        check_vma=False,
    )
    return fn(proj_vars, res_state, wr)


# speedup vs baseline: 9.0252x; 9.0252x over previous
import jax
import jax.numpy as jnp
from jax.experimental import pallas as pl
from jax.experimental.pallas import tpu as pltpu

LEAK = 0.6
BIAS = 1.6

_N_TILE = 512


def _esn_body(u_ref, s_ref, wr_ref, o_ref, s_bf16_ref):
    j = pl.program_id(0)

    @pl.when(j == 0)
    def _():
        s_bf16_ref[...] = s_ref[...].astype(jnp.bfloat16)

    pre = jax.lax.dot_general(
        s_bf16_ref[...],
        wr_ref[...].astype(jnp.bfloat16),
        dimension_numbers=(((1,), (1,)), ((), ())),
        preferred_element_type=jnp.float32,
    )
    pre = pre + u_ref[...] + BIAS
    s_tile = s_ref[:, pl.ds(j * _N_TILE, _N_TILE)]
    o_ref[...] = LEAK * jnp.tanh(pre) + (1.0 - LEAK) * s_tile


@jax.jit
def kernel(proj_vars, res_state, wr):
    batch, res_dim = res_state.shape
    n_tiles = wr.shape[0] // _N_TILE
    return pl.pallas_call(
        _esn_body,
        grid=(n_tiles,),
        in_specs=[
            pl.BlockSpec((batch, _N_TILE), lambda j: (0, j)),
            pl.BlockSpec((batch, res_dim), lambda j: (0, 0)),
            pl.BlockSpec((_N_TILE, res_dim), lambda j: (j, 0)),
        ],
        out_specs=pl.BlockSpec((batch, _N_TILE), lambda j: (0, j)),
        out_shape=jax.ShapeDtypeStruct((batch, wr.shape[0]), jnp.float32),
        scratch_shapes=[pltpu.VMEM((batch, res_dim), jnp.bfloat16)],
    )(proj_vars, res_state, wr)
